# baseline (speedup 1.0000x reference)
import functools

import jax
import jax.numpy as jnp
from jax import lax
from jax.experimental import pallas as pl
from jax.experimental.pallas import tpu as pltpu
from jax.experimental.pallas import tpu_sc as plsc

N = 10000
K = 32
D = 128

N_SC = 2560
N_TC = N - N_SC

BN = 496
BT = 320

NC, NS = 2, 16
NW = NC * NS
NPW = N_SC // NW
CH = 8
NCHUNK = NPW // CH
ROWS_CH = CH * K


def _sc_body(neigh_hbm, out_hbm, buf0, buf1, ob, sem0, sem1):
    c = lax.axis_index("c")
    s = lax.axis_index("s")
    wid = s * NC + c
    node0 = wid * NPW
    row0 = (N_TC + node0) * K
    bufs = (buf0, buf1)
    sems = (sem0, sem1)

    def start(ci):
        return pltpu.async_copy(
            neigh_hbm.at[pl.ds(row0 + ci * ROWS_CH, ROWS_CH)],
            bufs[ci % 2],
            sems[ci % 2],
        )

    cp = start(0)
    for ci in range(NCHUNK):
        nxt = start(ci + 1) if ci + 1 < NCHUNK else None
        cp.wait()
        buf = bufs[ci % 2]
        for j in range(CH):
            base = j * K

            def kbody(k, accs, _base=base, _buf=buf):
                return tuple(
                    accs[dc] + _buf[_base + k, pl.ds(dc * 16, 16)]
                    for dc in range(8)
                )

            accs = lax.fori_loop(
                0, K, kbody,
                tuple(jnp.zeros((16,), jnp.float32) for _ in range(8)),
            )
            for dc in range(8):
                ob[j, pl.ds(dc * 16, 16)] = accs[dc] * (1.0 / K)
        pltpu.sync_copy(ob, out_hbm.at[pl.ds(node0 + ci * CH, CH)])
        cp = nxt


_sc_agg = functools.partial(
    pl.kernel,
    out_type=jax.ShapeDtypeStruct((N_SC, D), jnp.float32),
    mesh=plsc.VectorSubcoreMesh(
        core_axis_name="c", subcore_axis_name="s",
        num_cores=NC, num_subcores=NS,
    ),
    scratch_types=[
        pltpu.VMEM((ROWS_CH, D), jnp.float32),
        pltpu.VMEM((ROWS_CH, D), jnp.float32),
        pltpu.VMEM((CH, D), jnp.float32),
        pltpu.SemaphoreType.DMA,
        pltpu.SemaphoreType.DMA,
    ],
)(_sc_body)


def _head_body(x_ref, nx_ref, ws_ref, wn_ref, o_ref):
    agg = jnp.sum(nx_ref[...], axis=1) * (1.0 / K)
    o_ref[...] = (
        jnp.dot(x_ref[...], ws_ref[...], preferred_element_type=jnp.float32)
        + jnp.dot(agg, wn_ref[...], preferred_element_type=jnp.float32)
    )


def _tail_body(x_ref, agg_ref, ws_ref, wn_ref, o_ref):
    o_ref[...] = (
        jnp.dot(x_ref[...], ws_ref[...], preferred_element_type=jnp.float32)
        + jnp.dot(agg_ref[...], wn_ref[...], preferred_element_type=jnp.float32)
    )


@jax.jit
def kernel(x, neigh_x, kernel_self, kernel_neigh):
    neigh_flat = neigh_x.reshape(N * K, D)

    agg_sc = _sc_agg(neigh_flat)

    head = pl.pallas_call(
        _head_body,
        grid=(N_TC // BN,),
        in_specs=[
            pl.BlockSpec((BN, D), lambda i: (i, 0)),
            pl.BlockSpec((BN, K, D), lambda i: (i, 0, 0)),
            pl.BlockSpec((D, D), lambda i: (0, 0)),
            pl.BlockSpec((D, D), lambda i: (0, 0)),
        ],
        out_specs=pl.BlockSpec((BN, D), lambda i: (i, 0)),
        out_shape=jax.ShapeDtypeStruct((N_TC, D), jnp.float32),
    )(x, neigh_x, kernel_self, kernel_neigh)

    x_tail = x[N_TC:]
    tail = pl.pallas_call(
        _tail_body,
        grid=(N_SC // BT,),
        in_specs=[
            pl.BlockSpec((BT, D), lambda i: (i, 0)),
            pl.BlockSpec((BT, D), lambda i: (i, 0)),
            pl.BlockSpec((D, D), lambda i: (0, 0)),
            pl.BlockSpec((D, D), lambda i: (0, 0)),
        ],
        out_specs=pl.BlockSpec((BT, D), lambda i: (i, 0)),
        out_shape=jax.ShapeDtypeStruct((N_SC, D), jnp.float32),
    )(x_tail, agg_sc, kernel_self, kernel_neigh)

    return jnp.concatenate([head, tail], axis=0)


# speedup vs baseline: 1.5526x; 1.5526x over previous
import jax
import jax.numpy as jnp
from jax.experimental import pallas as pl

N = 10000
K = 32
D = 128
BN = 400


def _body(x_ref, nx_ref, ws_ref, wn_ref, o_ref):
    agg = jnp.sum(nx_ref[...], axis=1) * (1.0 / K)
    o_ref[...] = (
        jnp.dot(x_ref[...], ws_ref[...], preferred_element_type=jnp.float32)
        + jnp.dot(agg, wn_ref[...], preferred_element_type=jnp.float32)
    )


@jax.jit
def kernel(x, neigh_x, kernel_self, kernel_neigh):
    return pl.pallas_call(
        _body,
        grid=(N // BN,),
        in_specs=[
            pl.BlockSpec((BN, D), lambda i: (i, 0)),
            pl.BlockSpec((BN, K, D), lambda i: (i, 0, 0)),
            pl.BlockSpec((D, D), lambda i: (0, 0)),
            pl.BlockSpec((D, D), lambda i: (0, 0)),
        ],
        out_specs=pl.BlockSpec((BN, D), lambda i: (i, 0)),
        out_shape=jax.ShapeDtypeStruct((N, D), jnp.float32),
    )(x, neigh_x, kernel_self, kernel_neigh)
